# baseline (speedup 1.0000x reference)
import functools
import math

import jax
import jax.numpy as jnp
import numpy as np
from jax import lax
from jax.experimental import pallas as pl
from jax.experimental.pallas import tpu as pltpu

N_NODES = 50000
N_EDGES = 800000
N_LG_EDGES = 1600000
EF = 32
AF = 32
BN_EPS = 1e-5

EBLK = 4000
NBLK = 1000


def _tables_body(x_ref, ws_ref, bs_ref, wd_ref, bd_ref, hs_ref, hd_ref):
    x = x_ref[...]
    hs_ref[...] = x @ ws_ref[...] + bs_ref[...]
    hd_ref[...] = x @ wd_ref[...] + bd_ref[...]


def _tables(x, ws, bs, wd, bd, blk):
    n, f = x.shape
    fo = ws.shape[1]
    grid = (n // blk,)
    return pl.pallas_call(
        _tables_body,
        grid=grid,
        in_specs=[
            pl.BlockSpec((blk, f), lambda i: (i, 0)),
            pl.BlockSpec((f, fo), lambda i: (0, 0)),
            pl.BlockSpec((1, fo), lambda i: (0, 0)),
            pl.BlockSpec((f, fo), lambda i: (0, 0)),
            pl.BlockSpec((1, fo), lambda i: (0, 0)),
        ],
        out_specs=[
            pl.BlockSpec((blk, fo), lambda i: (i, 0)),
            pl.BlockSpec((blk, fo), lambda i: (i, 0)),
        ],
        out_shape=[
            jax.ShapeDtypeStruct((n, fo), jnp.float32),
            jax.ShapeDtypeStruct((n, fo), jnp.float32),
        ],
    )(x, ws, bs[None, :], wd, bd[None, :])


def _linear_body(x_ref, w_ref, b_ref, o_ref):
    o_ref[...] = x_ref[...] @ w_ref[...] + b_ref[...]


def _linear(x, w, b, blk):
    n, f = x.shape
    fo = w.shape[1]
    return pl.pallas_call(
        _linear_body,
        grid=(n // blk,),
        in_specs=[
            pl.BlockSpec((blk, f), lambda i: (i, 0)),
            pl.BlockSpec((f, fo), lambda i: (0, 0)),
            pl.BlockSpec((1, fo), lambda i: (0, 0)),
        ],
        out_specs=pl.BlockSpec((blk, fo), lambda i: (i, 0)),
        out_shape=jax.ShapeDtypeStruct((n, fo), jnp.float32),
    )(x, w, b[None, :])


def _edge_body(a_ref, b_ref, ef_ref, we_ref, be_ref, g_ref, beta_ref,
               m2_ref, acc_ref, st_ref, *, nblocks, n_edges):
    ph = pl.program_id(0)
    i = pl.program_id(1)

    @pl.when((ph == 0) & (i == 0))
    def _():
        acc_ref[...] = jnp.zeros_like(acc_ref)

    m = a_ref[...] + b_ref[...] + ef_ref[...] @ we_ref[...] + be_ref[...]

    @pl.when(ph == 0)
    def _():
        acc_ref[0:1, :] += jnp.sum(m, axis=0, keepdims=True)
        acc_ref[1:2, :] += jnp.sum(m * m, axis=0, keepdims=True)

    @pl.when((ph == 0) & (i == nblocks - 1))
    def _():
        mu = acc_ref[0:1, :] / np.float32(n_edges)
        var = acc_ref[1:2, :] / np.float32(n_edges) - mu * mu
        s = g_ref[...] * lax.rsqrt(var + BN_EPS)
        st_ref[0:1, :] = s
        st_ref[1:2, :] = beta_ref[...] - mu * s

    @pl.when(ph == 1)
    def _():
        y = m * st_ref[0:1, :] + st_ref[1:2, :]
        nf = m2_ref.shape[1]
        yf = y[:, :nf]
        ys = y[:, nf:]
        m2_ref[...] = jax.nn.sigmoid(yf) * jax.nn.softplus(ys)


def _edge_stage(a, b, ef, we, be, g, beta, blk):
    e, f2 = a.shape
    fe = ef.shape[1]
    nf = f2 // 2
    nblocks = e // blk
    body = functools.partial(_edge_body, nblocks=nblocks, n_edges=e)
    return pl.pallas_call(
        body,
        grid=(2, nblocks),
        in_specs=[
            pl.BlockSpec((blk, f2), lambda p, i: (i, 0)),
            pl.BlockSpec((blk, f2), lambda p, i: (i, 0)),
            pl.BlockSpec((blk, fe), lambda p, i: (i, 0)),
            pl.BlockSpec((fe, f2), lambda p, i: (0, 0)),
            pl.BlockSpec((1, f2), lambda p, i: (0, 0)),
            pl.BlockSpec((1, f2), lambda p, i: (0, 0)),
            pl.BlockSpec((1, f2), lambda p, i: (0, 0)),
        ],
        out_specs=pl.BlockSpec((blk, nf), lambda p, i: (i, 0)),
        out_shape=jax.ShapeDtypeStruct((e, nf), jnp.float32),
        scratch_shapes=[
            pltpu.VMEM((2, f2), jnp.float32),
            pltpu.VMEM((2, f2), jnp.float32),
        ],
    )(a, b, ef, we, be[None, :], g[None, :], beta[None, :])


def _node_body(x_ref, p0_ref, p1_ref, g_ref, beta_ref, o_ref, acc_ref,
               st_ref, *, nblocks, n_nodes):
    ph = pl.program_id(0)
    i = pl.program_id(1)

    @pl.when((ph == 0) & (i == 0))
    def _():
        acc_ref[...] = jnp.zeros_like(acc_ref)

    h = p0_ref[...] + p1_ref[...]

    @pl.when(ph == 0)
    def _():
        acc_ref[0:1, :] += jnp.sum(h, axis=0, keepdims=True)
        acc_ref[1:2, :] += jnp.sum(h * h, axis=0, keepdims=True)

    @pl.when((ph == 0) & (i == nblocks - 1))
    def _():
        mu = acc_ref[0:1, :] / np.float32(n_nodes)
        var = acc_ref[1:2, :] / np.float32(n_nodes) - mu * mu
        s = g_ref[...] * lax.rsqrt(var + BN_EPS)
        st_ref[0:1, :] = s
        st_ref[1:2, :] = beta_ref[...] - mu * s

    @pl.when(ph == 1)
    def _():
        hn = h * st_ref[0:1, :] + st_ref[1:2, :]
        o_ref[...] = jax.nn.softplus(x_ref[...] + hn)


def _node_stage(x, p0, p1, g, beta, blk):
    n, nf = x.shape
    nblocks = n // blk
    body = functools.partial(_node_body, nblocks=nblocks, n_nodes=n)
    return pl.pallas_call(
        body,
        grid=(2, nblocks),
        in_specs=[
            pl.BlockSpec((blk, nf), lambda p, i: (i, 0)),
            pl.BlockSpec((blk, nf), lambda p, i: (i, 0)),
            pl.BlockSpec((blk, nf), lambda p, i: (i, 0)),
            pl.BlockSpec((1, nf), lambda p, i: (0, 0)),
            pl.BlockSpec((1, nf), lambda p, i: (0, 0)),
        ],
        out_specs=pl.BlockSpec((blk, nf), lambda p, i: (i, 0)),
        out_shape=jax.ShapeDtypeStruct((n, nf), jnp.float32),
        scratch_shapes=[
            pltpu.VMEM((2, nf), jnp.float32),
            pltpu.VMEM((2, nf), jnp.float32),
        ],
    )(x, p0, p1, g[None, :], beta[None, :])


def _gather_rows(table, idx):
    return jnp.take(table, idx, axis=0)


def _segment_partials(m2, dst, n_segments):
    h = jax.ops.segment_sum(m2, dst, num_segments=n_segments)
    return h, jnp.zeros_like(h)


def _head_body(n_feats_ref, fc_W_ref, fc_b_ref, out_W_ref, out_b_ref,
               o_ref, acc_ref):
    i = pl.program_id(0)
    ni = pl.num_programs(0)

    @pl.when(i == 0)
    def _():
        acc_ref[...] = jnp.zeros_like(acc_ref)

    acc_ref[...] += jnp.sum(n_feats_ref[...], axis=0, keepdims=True)

    @pl.when(i == ni - 1)
    def _():
        feats = acc_ref[...] / np.float32(N_NODES)
        feats = jax.nn.softplus(feats)
        feats = jax.nn.softplus(feats @ fc_W_ref[...] + fc_b_ref[...])
        feats = jax.nn.softplus(feats)
        o_ref[...] = feats @ out_W_ref[...] + out_b_ref[...]


def _head(n_feats, fc_W, fc_b, out_W, out_b):
    nf = n_feats.shape[1]
    out = pl.pallas_call(
        _head_body,
        grid=(N_NODES // NBLK,),
        in_specs=[
            pl.BlockSpec((NBLK, nf), lambda i: (i, 0)),
            pl.BlockSpec(fc_W.shape, lambda i: (0, 0)),
            pl.BlockSpec((1, fc_b.shape[0]), lambda i: (0, 0)),
            pl.BlockSpec(out_W.shape, lambda i: (0, 0)),
            pl.BlockSpec((1, 1), lambda i: (0, 0)),
        ],
        out_specs=pl.BlockSpec((1, 1), lambda i: (0, 0)),
        out_shape=jax.ShapeDtypeStruct((1, 1), jnp.float32),
        scratch_shapes=[pltpu.VMEM((1, nf), jnp.float32)],
    )(n_feats, fc_W, fc_b[None, :], out_W, out_b[None, :])
    return out[0, 0]


def _conv(x, e_feats, src, dst, n_nodes, p, nblk, eblk):
    hs, hd = _tables(x, p['src_W'], p['src_b'], p['dst_W'], p['dst_b'], nblk)
    a = _gather_rows(hs, src)
    b = _gather_rows(hd, dst)
    m2 = _edge_stage(a, b, e_feats, p['edge_W'], p['edge_b'],
                     p['bnm_g'], p['bnm_b'], eblk)
    p0, p1 = _segment_partials(m2, dst, n_nodes)
    return _node_stage(x, p0, p1, p['bn_g'], p['bn_b'], nblk)


def _rbf(d, vmin, vmax, bins):
    centers = jnp.linspace(vmin, vmax, bins)
    gamma = 1.0 / ((vmax - vmin) / (bins - 1))
    return jnp.exp(-gamma * (d[:, None] - centers) ** 2)


def kernel(atom_features, r, lg_angle, edge_index, lg_edge_index, params):
    L = params['c1']['src_W'].shape[0]
    bondlength = jnp.linalg.norm(r, axis=1)
    e_feats = _rbf(bondlength, 0.0, 8.0, EF)
    a_feats = _rbf(lg_angle, -np.pi / 2, np.pi / 2, AF)
    n_feats = _linear(atom_features, params['embed_W'], params['embed_b'],
                      NBLK)
    src, dst = edge_index[0], edge_index[1]
    lsrc, ldst = lg_edge_index[0], lg_edge_index[1]

    def layer(p, i):
        return {k: v[i] for k, v in p.items()}

    for i in range(L):
        n_feats = _conv(n_feats, e_feats, src, dst, N_NODES,
                        layer(params['c1'], i), NBLK, EBLK)
        e_feats = _conv(e_feats, a_feats, lsrc, ldst, N_EDGES,
                        layer(params['c2'], i), EBLK, EBLK * 2)

    return _head(n_feats, params['fc_W'], params['fc_b'],
                 params['out_W'], params['out_b'])


# speedup vs baseline: 1.2027x; 1.2027x over previous
import functools
import math

import jax
import jax.numpy as jnp
import numpy as np
from jax import lax
from jax.experimental import pallas as pl
from jax.experimental.pallas import tpu as pltpu
from jax.experimental.pallas import tpu_sc as plsc

N_NODES = 50000
N_EDGES = 800000
N_LG_EDGES = 1600000
EF = 32
AF = 32
BN_EPS = 1e-5

EBLK = 4000
NBLK = 1000


def _tables_body(x_ref, ws_ref, bs_ref, wd_ref, bd_ref, hs_ref, hd_ref):
    x = x_ref[...]
    hs_ref[...] = x @ ws_ref[...] + bs_ref[...]
    hd_ref[...] = x @ wd_ref[...] + bd_ref[...]


def _tables(x, ws, bs, wd, bd, blk):
    n, f = x.shape
    fo = ws.shape[1]
    grid = (n // blk,)
    return pl.pallas_call(
        _tables_body,
        grid=grid,
        in_specs=[
            pl.BlockSpec((blk, f), lambda i: (i, 0)),
            pl.BlockSpec((f, fo), lambda i: (0, 0)),
            pl.BlockSpec((1, fo), lambda i: (0, 0)),
            pl.BlockSpec((f, fo), lambda i: (0, 0)),
            pl.BlockSpec((1, fo), lambda i: (0, 0)),
        ],
        out_specs=[
            pl.BlockSpec((blk, fo), lambda i: (i, 0)),
            pl.BlockSpec((blk, fo), lambda i: (i, 0)),
        ],
        out_shape=[
            jax.ShapeDtypeStruct((n, fo), jnp.float32),
            jax.ShapeDtypeStruct((n, fo), jnp.float32),
        ],
    )(x, ws, bs[None, :], wd, bd[None, :])


def _linear_body(x_ref, w_ref, b_ref, o_ref):
    o_ref[...] = x_ref[...] @ w_ref[...] + b_ref[...]


def _linear(x, w, b, blk):
    n, f = x.shape
    fo = w.shape[1]
    return pl.pallas_call(
        _linear_body,
        grid=(n // blk,),
        in_specs=[
            pl.BlockSpec((blk, f), lambda i: (i, 0)),
            pl.BlockSpec((f, fo), lambda i: (0, 0)),
            pl.BlockSpec((1, fo), lambda i: (0, 0)),
        ],
        out_specs=pl.BlockSpec((blk, fo), lambda i: (i, 0)),
        out_shape=jax.ShapeDtypeStruct((n, fo), jnp.float32),
    )(x, w, b[None, :])


def _edge_body(a_ref, b_ref, ef_ref, we_ref, be_ref, g_ref, beta_ref,
               m2_ref, acc_ref, st_ref, *, nblocks, n_edges):
    ph = pl.program_id(0)
    i = pl.program_id(1)

    @pl.when((ph == 0) & (i == 0))
    def _():
        acc_ref[...] = jnp.zeros_like(acc_ref)

    m = a_ref[...] + b_ref[...] + ef_ref[...] @ we_ref[...] + be_ref[...]

    @pl.when(ph == 0)
    def _():
        acc_ref[0:1, :] += jnp.sum(m, axis=0, keepdims=True)
        acc_ref[1:2, :] += jnp.sum(m * m, axis=0, keepdims=True)

    @pl.when((ph == 0) & (i == nblocks - 1))
    def _():
        mu = acc_ref[0:1, :] / np.float32(n_edges)
        var = acc_ref[1:2, :] / np.float32(n_edges) - mu * mu
        s = g_ref[...] * lax.rsqrt(var + BN_EPS)
        st_ref[0:1, :] = s
        st_ref[1:2, :] = beta_ref[...] - mu * s

    @pl.when(ph == 1)
    def _():
        y = m * st_ref[0:1, :] + st_ref[1:2, :]
        nf = m2_ref.shape[1]
        yf = y[:, :nf]
        ys = y[:, nf:]
        m2_ref[...] = jax.nn.sigmoid(yf) * jax.nn.softplus(ys)


def _edge_stage(a, b, ef, we, be, g, beta, blk):
    e, f2 = a.shape
    fe = ef.shape[1]
    nf = f2 // 2
    nblocks = e // blk
    body = functools.partial(_edge_body, nblocks=nblocks, n_edges=e)
    return pl.pallas_call(
        body,
        grid=(2, nblocks),
        in_specs=[
            pl.BlockSpec((blk, f2), lambda p, i: (i, 0)),
            pl.BlockSpec((blk, f2), lambda p, i: (i, 0)),
            pl.BlockSpec((blk, fe), lambda p, i: (i, 0)),
            pl.BlockSpec((fe, f2), lambda p, i: (0, 0)),
            pl.BlockSpec((1, f2), lambda p, i: (0, 0)),
            pl.BlockSpec((1, f2), lambda p, i: (0, 0)),
            pl.BlockSpec((1, f2), lambda p, i: (0, 0)),
        ],
        out_specs=pl.BlockSpec((blk, nf), lambda p, i: (i, 0)),
        out_shape=jax.ShapeDtypeStruct((e, nf), jnp.float32),
        scratch_shapes=[
            pltpu.VMEM((2, f2), jnp.float32),
            pltpu.VMEM((2, f2), jnp.float32),
        ],
    )(a, b, ef, we, be[None, :], g[None, :], beta[None, :])


def _node_body(x_ref, p0_ref, p1_ref, g_ref, beta_ref, o_ref, acc_ref,
               st_ref, *, nblocks, n_nodes):
    ph = pl.program_id(0)
    i = pl.program_id(1)

    @pl.when((ph == 0) & (i == 0))
    def _():
        acc_ref[...] = jnp.zeros_like(acc_ref)

    h = p0_ref[...] + p1_ref[...]

    @pl.when(ph == 0)
    def _():
        acc_ref[0:1, :] += jnp.sum(h, axis=0, keepdims=True)
        acc_ref[1:2, :] += jnp.sum(h * h, axis=0, keepdims=True)

    @pl.when((ph == 0) & (i == nblocks - 1))
    def _():
        mu = acc_ref[0:1, :] / np.float32(n_nodes)
        var = acc_ref[1:2, :] / np.float32(n_nodes) - mu * mu
        s = g_ref[...] * lax.rsqrt(var + BN_EPS)
        st_ref[0:1, :] = s
        st_ref[1:2, :] = beta_ref[...] - mu * s

    @pl.when(ph == 1)
    def _():
        hn = h * st_ref[0:1, :] + st_ref[1:2, :]
        o_ref[...] = jax.nn.softplus(x_ref[...] + hn)


def _node_stage(x, p0, p1, g, beta, blk):
    n, nf = x.shape
    nblocks = n // blk
    body = functools.partial(_node_body, nblocks=nblocks, n_nodes=n)
    return pl.pallas_call(
        body,
        grid=(2, nblocks),
        in_specs=[
            pl.BlockSpec((blk, nf), lambda p, i: (i, 0)),
            pl.BlockSpec((blk, nf), lambda p, i: (i, 0)),
            pl.BlockSpec((blk, nf), lambda p, i: (i, 0)),
            pl.BlockSpec((1, nf), lambda p, i: (0, 0)),
            pl.BlockSpec((1, nf), lambda p, i: (0, 0)),
        ],
        out_specs=pl.BlockSpec((blk, nf), lambda p, i: (i, 0)),
        out_shape=jax.ShapeDtypeStruct((n, nf), jnp.float32),
        scratch_shapes=[
            pltpu.VMEM((2, nf), jnp.float32),
            pltpu.VMEM((2, nf), jnp.float32),
        ],
    )(x, p0, p1, g[None, :], beta[None, :])


_SC_MESH = dict(core_axis_name="c", subcore_axis_name="s")
_NW = 32
_GCH = 200


def _gather_body(hs_hbm, hd_hbm, src_hbm, dst_hbm, a_hbm, b_hbm,
                 sidx, didx, arows, brows, sema, semb, *, rows_per_w):
    wid = lax.axis_index("s") * 2 + lax.axis_index("c")
    w0 = wid * rows_per_w
    nch = rows_per_w // _GCH

    def step(k, _):
        base = w0 + k * _GCH
        pltpu.sync_copy(src_hbm.at[pl.ds(base, _GCH)], sidx)
        pltpu.sync_copy(dst_hbm.at[pl.ds(base, _GCH)], didx)
        ca = pltpu.make_async_copy(hs_hbm.at[sidx], arows, sema)
        cb = pltpu.make_async_copy(hd_hbm.at[didx], brows, semb)
        ca.start()
        cb.start()
        ca.wait()
        cb.wait()
        pltpu.sync_copy(arows, a_hbm.at[pl.ds(base, _GCH)])
        pltpu.sync_copy(brows, b_hbm.at[pl.ds(base, _GCH)])
        return 0

    lax.fori_loop(0, nch, step, 0)


def _sc_gather2(hs, hd, src, dst):
    e = src.shape[0]
    f = hs.shape[1]
    rows_per_w = e // _NW
    mesh = plsc.VectorSubcoreMesh(**_SC_MESH)
    body = functools.partial(_gather_body, rows_per_w=rows_per_w)
    k = pl.kernel(
        body,
        mesh=mesh,
        compiler_params=pltpu.CompilerParams(use_tc_tiling_on_sc=False),
        out_type=[
            jax.ShapeDtypeStruct((e, f), jnp.float32),
            jax.ShapeDtypeStruct((e, f), jnp.float32),
        ],
        scratch_types=[
            pltpu.VMEM((_GCH,), jnp.int32),
            pltpu.VMEM((_GCH,), jnp.int32),
            pltpu.VMEM((_GCH, f), jnp.float32),
            pltpu.VMEM((_GCH, f), jnp.float32),
            pltpu.SemaphoreType.DMA,
            pltpu.SemaphoreType.DMA,
        ],
    )
    return k(hs, hd, src, dst)


def _gather_rows(table, idx):
    return jnp.take(table, idx, axis=0)


def _segment_partials(m2, dst, n_segments):
    h = jax.ops.segment_sum(m2, dst, num_segments=n_segments)
    return h, jnp.zeros_like(h)


def _head_body(n_feats_ref, fc_W_ref, fc_b_ref, out_W_ref, out_b_ref,
               o_ref, acc_ref):
    i = pl.program_id(0)
    ni = pl.num_programs(0)

    @pl.when(i == 0)
    def _():
        acc_ref[...] = jnp.zeros_like(acc_ref)

    acc_ref[...] += jnp.sum(n_feats_ref[...], axis=0, keepdims=True)

    @pl.when(i == ni - 1)
    def _():
        feats = acc_ref[...] / np.float32(N_NODES)
        feats = jax.nn.softplus(feats)
        feats = jax.nn.softplus(feats @ fc_W_ref[...] + fc_b_ref[...])
        feats = jax.nn.softplus(feats)
        o_ref[...] = feats @ out_W_ref[...] + out_b_ref[...]


def _head(n_feats, fc_W, fc_b, out_W, out_b):
    nf = n_feats.shape[1]
    out = pl.pallas_call(
        _head_body,
        grid=(N_NODES // NBLK,),
        in_specs=[
            pl.BlockSpec((NBLK, nf), lambda i: (i, 0)),
            pl.BlockSpec(fc_W.shape, lambda i: (0, 0)),
            pl.BlockSpec((1, fc_b.shape[0]), lambda i: (0, 0)),
            pl.BlockSpec(out_W.shape, lambda i: (0, 0)),
            pl.BlockSpec((1, 1), lambda i: (0, 0)),
        ],
        out_specs=pl.BlockSpec((1, 1), lambda i: (0, 0)),
        out_shape=jax.ShapeDtypeStruct((1, 1), jnp.float32),
        scratch_shapes=[pltpu.VMEM((1, nf), jnp.float32)],
    )(n_feats, fc_W, fc_b[None, :], out_W, out_b[None, :])
    return out[0, 0]


def _conv(x, e_feats, src, dst, n_nodes, p, nblk, eblk):
    hs, hd = _tables(x, p['src_W'], p['src_b'], p['dst_W'], p['dst_b'], nblk)
    a, b = _sc_gather2(hs, hd, src, dst)
    m2 = _edge_stage(a, b, e_feats, p['edge_W'], p['edge_b'],
                     p['bnm_g'], p['bnm_b'], eblk)
    p0, p1 = _segment_partials(m2, dst, n_nodes)
    return _node_stage(x, p0, p1, p['bn_g'], p['bn_b'], nblk)


def _rbf(d, vmin, vmax, bins):
    centers = jnp.linspace(vmin, vmax, bins)
    gamma = 1.0 / ((vmax - vmin) / (bins - 1))
    return jnp.exp(-gamma * (d[:, None] - centers) ** 2)


def kernel(atom_features, r, lg_angle, edge_index, lg_edge_index, params):
    L = params['c1']['src_W'].shape[0]
    bondlength = jnp.linalg.norm(r, axis=1)
    e_feats = _rbf(bondlength, 0.0, 8.0, EF)
    a_feats = _rbf(lg_angle, -np.pi / 2, np.pi / 2, AF)
    n_feats = _linear(atom_features, params['embed_W'], params['embed_b'],
                      NBLK)
    src, dst = edge_index[0], edge_index[1]
    lsrc, ldst = lg_edge_index[0], lg_edge_index[1]

    def layer(p, i):
        return {k: v[i] for k, v in p.items()}

    for i in range(L):
        n_feats = _conv(n_feats, e_feats, src, dst, N_NODES,
                        layer(params['c1'], i), NBLK, EBLK)
        e_feats = _conv(e_feats, a_feats, lsrc, ldst, N_EDGES,
                        layer(params['c2'], i), EBLK, EBLK * 2)

    return _head(n_feats, params['fc_W'], params['fc_b'],
                 params['out_W'], params['out_b'])


# speedup vs baseline: 1.2623x; 1.0495x over previous
import functools
import math

import jax
import jax.numpy as jnp
import numpy as np
from jax import lax
from jax.experimental import pallas as pl
from jax.experimental.pallas import tpu as pltpu
from jax.experimental.pallas import tpu_sc as plsc

N_NODES = 50000
N_EDGES = 800000
N_LG_EDGES = 1600000
EF = 32
AF = 32
BN_EPS = 1e-5

EBLK = 4000
NBLK = 1000


def _dot(a, b):
    return jax.lax.dot(a, b, precision=jax.lax.Precision.DEFAULT)


def _tables_body(x_ref, ws_ref, bs_ref, wd_ref, bd_ref, hs_ref, hd_ref):
    x = x_ref[...]
    hs_ref[...] = _dot(x, ws_ref[...]) + bs_ref[...]
    hd_ref[...] = _dot(x, wd_ref[...]) + bd_ref[...]


def _tables(x, ws, bs, wd, bd, blk):
    n, f = x.shape
    fo = ws.shape[1]
    grid = (n // blk,)
    return pl.pallas_call(
        _tables_body,
        grid=grid,
        in_specs=[
            pl.BlockSpec((blk, f), lambda i: (i, 0)),
            pl.BlockSpec((f, fo), lambda i: (0, 0)),
            pl.BlockSpec((1, fo), lambda i: (0, 0)),
            pl.BlockSpec((f, fo), lambda i: (0, 0)),
            pl.BlockSpec((1, fo), lambda i: (0, 0)),
        ],
        out_specs=[
            pl.BlockSpec((blk, fo), lambda i: (i, 0)),
            pl.BlockSpec((blk, fo), lambda i: (i, 0)),
        ],
        out_shape=[
            jax.ShapeDtypeStruct((n, fo), jnp.float32),
            jax.ShapeDtypeStruct((n, fo), jnp.float32),
        ],
    )(x, ws, bs[None, :], wd, bd[None, :])


def _linear_body(x_ref, w_ref, b_ref, o_ref):
    o_ref[...] = _dot(x_ref[...], w_ref[...]) + b_ref[...]


def _linear(x, w, b, blk):
    n, f = x.shape
    fo = w.shape[1]
    return pl.pallas_call(
        _linear_body,
        grid=(n // blk,),
        in_specs=[
            pl.BlockSpec((blk, f), lambda i: (i, 0)),
            pl.BlockSpec((f, fo), lambda i: (0, 0)),
            pl.BlockSpec((1, fo), lambda i: (0, 0)),
        ],
        out_specs=pl.BlockSpec((blk, fo), lambda i: (i, 0)),
        out_shape=jax.ShapeDtypeStruct((n, fo), jnp.float32),
    )(x, w, b[None, :])


def _edge_body(a_ref, b_ref, ef_ref, we_ref, be_ref, g_ref, beta_ref,
               m2_ref, acc_ref, cmp_ref, mu0_ref, st_ref, *, nblocks, n_edges):
    ph = pl.program_id(0)
    i = pl.program_id(1)

    @pl.when((ph == 0) & (i == 0))
    def _():
        acc_ref[...] = jnp.zeros_like(acc_ref)
        cmp_ref[...] = jnp.zeros_like(cmp_ref)

    m = a_ref[...] + b_ref[...] + _dot(ef_ref[...], we_ref[...]) + be_ref[...]

    @pl.when((ph == 0) & (i == 0))
    def _():
        mu0_ref[...] = jnp.mean(m, axis=0, keepdims=True)

    @pl.when(ph == 0)
    def _():
        d = m - mu0_ref[...]
        s = jnp.concatenate(
            [jnp.sum(d, axis=0, keepdims=True),
             jnp.sum(d * d, axis=0, keepdims=True)], axis=0)
        y = s - cmp_ref[...]
        t = acc_ref[...] + y
        cmp_ref[...] = (t - acc_ref[...]) - y
        acc_ref[...] = t

    @pl.when((ph == 0) & (i == nblocks - 1))
    def _():
        md = acc_ref[0:1, :] / np.float32(n_edges)
        var = acc_ref[1:2, :] / np.float32(n_edges) - md * md
        mu = mu0_ref[...] + md
        s = g_ref[...] * lax.rsqrt(var + BN_EPS)
        st_ref[0:1, :] = s
        st_ref[1:2, :] = beta_ref[...] - mu * s

    @pl.when(ph == 1)
    def _():
        y = m * st_ref[0:1, :] + st_ref[1:2, :]
        nf = y.shape[1] // 2
        nfh = nf // 2
        m2 = jax.nn.sigmoid(y[:, :nf]) * jax.nn.softplus(y[:, nf:])
        m2_ref[0, :, :] = m2[:, :nfh]
        m2_ref[1, :, :] = m2[:, nfh:]


def _edge_stage(a, b, ef, we, be, g, beta, blk):
    e, f2 = a.shape
    fe = ef.shape[1]
    nf = f2 // 2
    nblocks = e // blk
    body = functools.partial(_edge_body, nblocks=nblocks, n_edges=e)
    return pl.pallas_call(
        body,
        grid=(2, nblocks),
        in_specs=[
            pl.BlockSpec((blk, f2), lambda p, i: (i, 0)),
            pl.BlockSpec((blk, f2), lambda p, i: (i, 0)),
            pl.BlockSpec((blk, fe), lambda p, i: (i, 0)),
            pl.BlockSpec((fe, f2), lambda p, i: (0, 0)),
            pl.BlockSpec((1, f2), lambda p, i: (0, 0)),
            pl.BlockSpec((1, f2), lambda p, i: (0, 0)),
            pl.BlockSpec((1, f2), lambda p, i: (0, 0)),
        ],
        out_specs=pl.BlockSpec((2, blk, nf // 2), lambda p, i: (0, i, 0)),
        out_shape=jax.ShapeDtypeStruct((2, e, nf // 2), jnp.float32),
        scratch_shapes=[
            pltpu.VMEM((2, f2), jnp.float32),
            pltpu.VMEM((2, f2), jnp.float32),
            pltpu.VMEM((1, f2), jnp.float32),
            pltpu.VMEM((2, f2), jnp.float32),
        ],
    )(a, b, ef, we, be[None, :], g[None, :], beta[None, :])


def _node_body(x_ref, p0_ref, p1_ref, g_ref, beta_ref, o_ref, acc_ref,
               cmp_ref, mu0_ref, st_ref, *, nblocks, n_nodes):
    ph = pl.program_id(0)
    i = pl.program_id(1)

    @pl.when((ph == 0) & (i == 0))
    def _():
        acc_ref[...] = jnp.zeros_like(acc_ref)
        cmp_ref[...] = jnp.zeros_like(cmp_ref)

    h = jnp.concatenate([p0_ref[0], p1_ref[0]], axis=1)

    @pl.when((ph == 0) & (i == 0))
    def _():
        mu0_ref[...] = jnp.mean(h, axis=0, keepdims=True)

    @pl.when(ph == 0)
    def _():
        d = h - mu0_ref[...]
        s = jnp.concatenate(
            [jnp.sum(d, axis=0, keepdims=True),
             jnp.sum(d * d, axis=0, keepdims=True)], axis=0)
        y = s - cmp_ref[...]
        t = acc_ref[...] + y
        cmp_ref[...] = (t - acc_ref[...]) - y
        acc_ref[...] = t

    @pl.when((ph == 0) & (i == nblocks - 1))
    def _():
        md = acc_ref[0:1, :] / np.float32(n_nodes)
        var = acc_ref[1:2, :] / np.float32(n_nodes) - md * md
        mu = mu0_ref[...] + md
        s = g_ref[...] * lax.rsqrt(var + BN_EPS)
        st_ref[0:1, :] = s
        st_ref[1:2, :] = beta_ref[...] - mu * s

    @pl.when(ph == 1)
    def _():
        hn = h * st_ref[0:1, :] + st_ref[1:2, :]
        o_ref[...] = x_ref[...] + hn


def _node_stage(x, hp, g, beta, blk):
    n, nf = x.shape
    nfh = nf // 2
    nblocks = n // blk
    body = functools.partial(_node_body, nblocks=nblocks, n_nodes=n)
    return pl.pallas_call(
        body,
        grid=(2, nblocks),
        in_specs=[
            pl.BlockSpec((blk, nf), lambda p, i: (i, 0)),
            pl.BlockSpec((1, blk, nfh), lambda p, i: (0, i, 0)),
            pl.BlockSpec((1, blk, nfh), lambda p, i: (1, i, 0)),
            pl.BlockSpec((1, nf), lambda p, i: (0, 0)),
            pl.BlockSpec((1, nf), lambda p, i: (0, 0)),
        ],
        out_specs=pl.BlockSpec((blk, nf), lambda p, i: (i, 0)),
        out_shape=jax.ShapeDtypeStruct((n, nf), jnp.float32),
        scratch_shapes=[
            pltpu.VMEM((2, nf), jnp.float32),
            pltpu.VMEM((2, nf), jnp.float32),
            pltpu.VMEM((1, nf), jnp.float32),
            pltpu.VMEM((2, nf), jnp.float32),
        ],
    )(x, hp, hp, g[None, :], beta[None, :])


_SC_MESH = dict(core_axis_name="c", subcore_axis_name="s")
_NW = 32
_GCH = 200


def _gather_body(hs_hbm, hd_hbm, src_hbm, dst_hbm, a_hbm, b_hbm,
                 sidx, didx, arows, brows, sema, semb, *, rows_per_w):
    wid = lax.axis_index("s") * 2 + lax.axis_index("c")
    w0 = wid * rows_per_w
    nch = rows_per_w // _GCH

    def step(k, _):
        base = w0 + k * _GCH
        pltpu.sync_copy(src_hbm.at[pl.ds(base, _GCH)], sidx)
        pltpu.sync_copy(dst_hbm.at[pl.ds(base, _GCH)], didx)
        ca = pltpu.make_async_copy(hs_hbm.at[sidx], arows, sema)
        cb = pltpu.make_async_copy(hd_hbm.at[didx], brows, semb)
        ca.start()
        cb.start()
        ca.wait()
        cb.wait()
        pltpu.sync_copy(arows, a_hbm.at[pl.ds(base, _GCH)])
        pltpu.sync_copy(brows, b_hbm.at[pl.ds(base, _GCH)])
        return 0

    lax.fori_loop(0, nch, step, 0)


def _sc_gather2(hs, hd, src, dst):
    e = src.shape[0]
    f = hs.shape[1]
    rows_per_w = e // _NW
    mesh = plsc.VectorSubcoreMesh(**_SC_MESH)
    body = functools.partial(_gather_body, rows_per_w=rows_per_w)
    k = pl.kernel(
        body,
        mesh=mesh,
        compiler_params=pltpu.CompilerParams(use_tc_tiling_on_sc=False),
        out_type=[
            jax.ShapeDtypeStruct((e, f), jnp.float32),
            jax.ShapeDtypeStruct((e, f), jnp.float32),
        ],
        scratch_types=[
            pltpu.VMEM((_GCH,), jnp.int32),
            pltpu.VMEM((_GCH,), jnp.int32),
            pltpu.VMEM((_GCH, f), jnp.float32),
            pltpu.VMEM((_GCH, f), jnp.float32),
            pltpu.SemaphoreType.DMA,
            pltpu.SemaphoreType.DMA,
        ],
    )
    return k(hs, hd, src, dst)


def _gather_rows(table, idx):
    return jnp.take(table, idx, axis=0)


_CB = 1280
_ZB = 800


def _scatter_body(m2_hbm, dst_hbm, z_hbm, out_hbm,
                  ids_v, idxp, rows_v, zbuf, table_sh,
                  *, e, rs, rs_pad, nfh, nranges):
    cid = lax.axis_index("c")
    sid = lax.axis_index("s")
    wid = sid * 2 + cid
    stripe = rs_pad // 16
    zc = stripe // _ZB
    tch = e // _CB

    pltpu.sync_copy(z_hbm, zbuf)
    nk = (tch - wid + 31) // 32

    for r in range(nranges):
        base = r * rs
        for z in range(zc):
            pltpu.sync_copy(zbuf,
                            table_sh.at[pl.ds(sid * stripe + z * _ZB, _ZB)])
        plsc.subcore_barrier()

        def chunk(k, _):
            cb = (wid + k * 32) * _CB
            pltpu.sync_copy(dst_hbm.at[pl.ds(cb, _CB)], ids_v)
            pltpu.sync_copy(m2_hbm.at[cid].at[pl.ds(cb, _CB)], rows_v)
            for j in range(_CB // 128):
                for g in range(8):
                    v = ids_v[pl.ds(j * 128 + g * 16, 16)]
                    rel = v - base
                    ok = (rel >= 0) & (rel < rs)
                    tr = rs + (v & 511)
                    idxp[pl.ds(g * 16, 16)] = jnp.where(ok, rel, tr)
                pltpu.sync_copy(rows_v.at[pl.ds(j * 128, 128)],
                                table_sh.at[idxp],
                                add=True)
            return 0

        lax.fori_loop(0, nk, chunk, 0)
        plsc.subcore_barrier()
        for w in range(zc):
            off = sid * stripe + w * _ZB
            pltpu.sync_copy(table_sh.at[pl.ds(off, _ZB)],
                            out_hbm.at[cid].at[pl.ds(base + off, _ZB)])
        plsc.subcore_barrier()


def _sc_scatter(m2s, dst, n_segments, rs, rs_pad):
    _, e, nfh = m2s.shape
    nranges = n_segments // rs
    mesh = plsc.VectorSubcoreMesh(**_SC_MESH)
    body = functools.partial(_scatter_body, e=e, rs=rs, rs_pad=rs_pad,
                             nfh=nfh, nranges=nranges)
    k = pl.kernel(
        body,
        mesh=mesh,
        compiler_params=pltpu.CompilerParams(use_tc_tiling_on_sc=False),
        out_type=jax.ShapeDtypeStruct(
            (2, n_segments + rs_pad - rs, nfh), jnp.float32),
        scratch_types=[
            pltpu.VMEM((_CB,), jnp.int32),
            pltpu.VMEM((128,), jnp.int32),
            pltpu.VMEM((_CB, nfh), jnp.float32),
            pltpu.VMEM((_ZB, nfh), jnp.float32),
            pltpu.VMEM_SHARED((rs_pad, nfh), jnp.float32),
        ],
    )
    zrows = jnp.zeros((_ZB, nfh), jnp.float32)
    return k(m2s, dst, zrows)


def _head_body(n_feats_ref, o_ref, acc_ref, cmp_ref):
    i = pl.program_id(0)
    ni = pl.num_programs(0)

    @pl.when(i == 0)
    def _():
        acc_ref[...] = jnp.zeros_like(acc_ref)
        cmp_ref[...] = jnp.zeros_like(cmp_ref)

    s = jnp.sum(n_feats_ref[...], axis=0, keepdims=True)
    y = s - cmp_ref[...]
    t = acc_ref[...] + y
    cmp_ref[...] = (t - acc_ref[...]) - y
    acc_ref[...] = t

    @pl.when(i == ni - 1)
    def _():
        o_ref[...] = acc_ref[...] / np.float32(N_NODES)


def _mean_nodes(n_feats):
    nf = n_feats.shape[1]
    out = pl.pallas_call(
        _head_body,
        grid=(N_NODES // NBLK,),
        in_specs=[pl.BlockSpec((NBLK, nf), lambda i: (i, 0))],
        out_specs=pl.BlockSpec((1, nf), lambda i: (0, 0)),
        out_shape=jax.ShapeDtypeStruct((1, nf), jnp.float32),
        scratch_shapes=[pltpu.VMEM((1, nf), jnp.float32),
                        pltpu.VMEM((1, nf), jnp.float32)],
    )(n_feats)
    return out[0]


def _conv(x, e_feats, src, dst, n_nodes, p, nblk, eblk, rs, rs_pad):
    hs, hd = _tables(x, p['src_W'], p['src_b'], p['dst_W'], p['dst_b'], nblk)
    a, b = _sc_gather2(hs, hd, src, dst)
    m2s = _edge_stage(a, b, e_feats, p['edge_W'], p['edge_b'],
                      p['bnm_g'], p['bnm_b'], eblk)
    hp = _sc_scatter(m2s, dst, n_nodes, rs, rs_pad)
    pre = _node_stage(x, hp, p['bn_g'], p['bn_b'], nblk)
    return jax.nn.softplus(pre)


def _rbf(d, vmin, vmax, bins):
    centers = jnp.linspace(vmin, vmax, bins)
    gamma = 1.0 / ((vmax - vmin) / (bins - 1))
    return jnp.exp(-gamma * (d[:, None] - centers) ** 2)


def kernel(atom_features, r, lg_angle, edge_index, lg_edge_index, params):
    L = params['c1']['src_W'].shape[0]
    bondlength = jnp.linalg.norm(r, axis=1)
    e_feats = _rbf(bondlength, 0.0, 8.0, EF)
    a_feats = _rbf(lg_angle, -np.pi / 2, np.pi / 2, AF)
    n_feats = _linear(atom_features, params['embed_W'], params['embed_b'],
                      NBLK)
    src, dst = edge_index[0], edge_index[1]
    lsrc, ldst = lg_edge_index[0], lg_edge_index[1]

    def layer(p, i):
        return {k: v[i] for k, v in p.items()}

    for i in range(L):
        n_feats = _conv(n_feats, e_feats, src, dst, N_NODES,
                        layer(params['c1'], i), NBLK, EBLK,
                        rs=25000, rs_pad=25600)
        e_feats = _conv(e_feats, a_feats, lsrc, ldst, N_EDGES,
                        layer(params['c2'], i), EBLK, EBLK,
                        rs=50000, rs_pad=51200)

    f = jax.nn.softplus(_mean_nodes(n_feats))
    f = jax.nn.softplus(f @ params['fc_W'] + params['fc_b'])
    f = jax.nn.softplus(f)
    return jnp.squeeze(f @ params['out_W'] + params['out_b'])
